# baseline (speedup 1.0000x reference)
import functools

import jax
import jax.numpy as jnp
from jax import lax
from jax.experimental import pallas as pl
from jax.experimental.pallas import tpu as pltpu
from jax.experimental.pallas import tpu_sc as plsc

N = 10000
E = 320000
D_IN = 128
D_H = 256
NPAD = 10240

NC = 2
NS = 16
C = 125
EROWS = E // C
ROWS1 = EROWS // (NC * NS)
ROWS2 = EROWS // NS

GLEN = 16

ROW_BLK = 1000
GRID = N // ROW_BLK

_sc_mesh = plsc.VectorSubcoreMesh(
    core_axis_name="c", subcore_axis_name="s", num_cores=NC, num_subcores=NS)


@functools.partial(
    pl.kernel,
    out_type=[
        jax.ShapeDtypeStruct((2 * N, D_IN), jnp.float32),
    ],
    mesh=_sc_mesh,
    scratch_types=[
        pltpu.VMEM((2 * GLEN, C), jnp.int32),
        pltpu.VMEM((2 * GLEN, C), jnp.int32),
        pltpu.VMEM((C, D_IN), jnp.float32),
        pltpu.VMEM((C, D_IN), jnp.float32),
        pltpu.VMEM_SHARED((N, D_IN), jnp.float32),
        pltpu.SemaphoreType.DMA,
        pltpu.SemaphoreType.DMA,
        pltpu.SemaphoreType.DMA,
        pltpu.SemaphoreType.DMA,
        pltpu.SemaphoreType.DMA,
    ],
)
def _sc_agg1(x_hbm, srcs_hbm, dsts_hbm, zeros_hbm, out_hbm,
             src_v, dst_v, rows0, rows1, acc_sh,
             gsem0, gsem1, ssem0, ssem1, isem):
    c = lax.axis_index("c")
    s = lax.axis_index("s")

    _par_init(zeros_hbm, acc_sh, s)

    base = c * (NS * ROWS1) + s * ROWS1
    pltpu.sync_copy(srcs_hbm.at[pl.ds(base, GLEN)], src_v.at[pl.ds(0, GLEN)])
    pltpu.sync_copy(dsts_hbm.at[pl.ds(base, GLEN)], dst_v.at[pl.ds(0, GLEN)])
    plsc.subcore_barrier()
    _agg_pipeline(x_hbm, srcs_hbm, dsts_hbm, base,
                  ROWS1 // GLEN, src_v, dst_v, (rows0, rows1),
                  (gsem0, gsem1), (ssem0, ssem1), isem, acc_sh)
    plsc.subcore_barrier()

    _par_writeout(acc_sh, out_hbm, c, s)


def _par_init(zeros_hbm, acc_sh, s):
    pltpu.sync_copy(zeros_hbm.at[pl.ds(s * 624, 624)],
                    acc_sh.at[pl.ds(s * 624, 624)])

    @pl.when(s == 0)
    def _tail():
        pltpu.sync_copy(zeros_hbm.at[pl.ds(16 * 624, N - 16 * 624)],
                        acc_sh.at[pl.ds(16 * 624, N - 16 * 624)])


def _par_writeout(acc_sh, out_hbm, c, s):
    pltpu.sync_copy(acc_sh.at[pl.ds(s * 624, 624)],
                    out_hbm.at[pl.ds(c * N + s * 624, 624)])

    @pl.when(s == 0)
    def _tail():
        pltpu.sync_copy(acc_sh.at[pl.ds(16 * 624, N - 16 * 624)],
                        out_hbm.at[pl.ds(c * N + 16 * 624, N - 16 * 624)])


def _agg_pipeline(tab_hbm, srcs_ref, dsts_ref, base, ng, src_v, dst_v,
                  rows, gsem, ssem, isem, acc_sh):
    def swait(p):
        pltpu.make_async_copy(rows[p], acc_sh.at[dst_v.at[0]], ssem[p]).wait()

    def group(g, carry):
        gp = lax.rem(g, 2)
        row0 = gp * GLEN

        @pl.when(g + 1 < ng)
        def _prefetch_idx():
            pltpu.async_copy(srcs_ref.at[pl.ds(base + (g + 1) * GLEN, GLEN)],
                             src_v.at[pl.ds((1 - gp) * GLEN, GLEN)], isem)
            pltpu.async_copy(dsts_ref.at[pl.ds(base + (g + 1) * GLEN, GLEN)],
                             dst_v.at[pl.ds((1 - gp) * GLEN, GLEN)], isem)

        @pl.when(g > 0)
        def _drain0():
            swait(0)

        gd = [pltpu.async_copy(tab_hbm.at[src_v.at[row0]], rows[0], gsem[0]),
              None]
        for jj in range(GLEN):
            p = jj % 2
            if jj < GLEN - 1:
                if jj >= 1:
                    swait(1 - p)
                else:
                    @pl.when(g > 0)
                    def _drain1():
                        swait(1)
                gd[1 - p] = pltpu.async_copy(
                    tab_hbm.at[src_v.at[row0 + jj + 1]], rows[1 - p],
                    gsem[1 - p])
            gd[p].wait()
            pltpu.async_copy(
                rows[p], acc_sh.at[dst_v.at[row0 + jj]], ssem[p], add=True)

        @pl.when(g + 1 < ng)
        def _wait_idx():
            pltpu.make_async_copy(srcs_ref.at[pl.ds(base, GLEN)],
                                  src_v.at[pl.ds((1 - gp) * GLEN, GLEN)],
                                  isem).wait()
            pltpu.make_async_copy(dsts_ref.at[pl.ds(base, GLEN)],
                                  dst_v.at[pl.ds((1 - gp) * GLEN, GLEN)],
                                  isem).wait()
        return carry

    lax.fori_loop(0, ng, group, 0)
    swait(0)
    swait(1)


@functools.partial(
    pl.kernel,
    out_type=[jax.ShapeDtypeStruct((2, NPAD), jnp.float32)],
    mesh=_sc_mesh,
    compiler_params=pltpu.CompilerParams(needs_layout_passes=False),
    scratch_types=[
        pltpu.VMEM((4000,), jnp.int32),
        pltpu.VMEM((NPAD,), jnp.float32),
        pltpu.VMEM((NS * 320,), jnp.float32),
        pltpu.VMEM((640,), jnp.float32),
        pltpu.VMEM_SHARED((NS * NPAD,), jnp.float32),
    ],
)
def _sc_deg(dstf_hbm, zerosd_hbm, deg_hbm, dfl_v, deg_t, red_v, dout_v,
            degs_sh):
    c = lax.axis_index("c")
    s = lax.axis_index("s")
    pltpu.sync_copy(zerosd_hbm, deg_t)

    base = c * (E // 2) + s * 10000
    for part, size in ((0, 4000), (4000, 4000), (8000, 2000)):
        pltpu.sync_copy(dstf_hbm.at[pl.ds(base + part, size)],
                        dfl_v.at[pl.ds(0, size)])

        def hist(i, carry):
            idx16 = dfl_v[pl.ds(i * 16, 16)]
            cnt, last = plsc.scan_count(idx16)
            plsc.addupdate_scatter(
                deg_t, [idx16], cnt.astype(jnp.float32), mask=last)
            return carry

        lax.fori_loop(0, size // 16, hist, 0)
    pltpu.sync_copy(deg_t, degs_sh.at[pl.ds(s * NPAD, NPAD)])
    plsc.subcore_barrier()

    for half in range(2):
        col0 = s * 640 + half * 320
        for r in range(NS):
            pltpu.sync_copy(degs_sh.at[pl.ds(r * NPAD + col0, 320)],
                            red_v.at[pl.ds(r * 320, 320)])

        def red(k, carry):
            acc16 = red_v[pl.ds(k * 16, 16)]
            for r in range(1, NS):
                acc16 = acc16 + red_v[pl.ds(r * 320 + k * 16, 16)]
            dout_v[pl.ds(half * 320 + k * 16, 16)] = acc16
            return carry

        lax.fori_loop(0, 320 // 16, red, 0)
    pltpu.sync_copy(dout_v, deg_hbm.at[c, pl.ds(s * 640, 640)])


@functools.partial(
    pl.kernel,
    out_type=[
        jax.ShapeDtypeStruct((2 * N, D_IN), jnp.float32),
    ],
    mesh=_sc_mesh,
    scratch_types=[
        pltpu.VMEM((2 * GLEN, C), jnp.int32),
        pltpu.VMEM((2 * GLEN, C), jnp.int32),
        pltpu.VMEM((C, D_IN), jnp.float32),
        pltpu.VMEM((C, D_IN), jnp.float32),
        pltpu.VMEM_SHARED((N, D_IN), jnp.float32),
        pltpu.SemaphoreType.DMA,
        pltpu.SemaphoreType.DMA,
        pltpu.SemaphoreType.DMA,
        pltpu.SemaphoreType.DMA,
        pltpu.SemaphoreType.DMA,
    ],
)
def _sc_agg2(ht_hbm, srcs2_hbm, dsts_hbm, zeros_hbm, out_hbm,
             src_v, dst_v, rows0, rows1, acc_sh,
             gsem0, gsem1, ssem0, ssem1, isem):
    c = lax.axis_index("c")
    s = lax.axis_index("s")

    _par_init(zeros_hbm, acc_sh, s)

    base = s * ROWS2
    srcs_ref = srcs2_hbm.at[c]
    pltpu.sync_copy(srcs_ref.at[pl.ds(base, GLEN)], src_v.at[pl.ds(0, GLEN)])
    pltpu.sync_copy(dsts_hbm.at[pl.ds(base, GLEN)], dst_v.at[pl.ds(0, GLEN)])
    plsc.subcore_barrier()
    _agg_pipeline(ht_hbm, srcs_ref, dsts_hbm, base,
                  ROWS2 // GLEN, src_v, dst_v, (rows0, rows1),
                  (gsem0, gsem1), (ssem0, ssem1), isem, acc_sh)
    plsc.subcore_barrier()

    _par_writeout(acc_sh, out_hbm, c, s)


def _self1_body(x_ref, wr_ref, b_ref, z_ref):
    z_ref[...] = (jnp.dot(x_ref[...], wr_ref[...],
                          preferred_element_type=jnp.float32) + b_ref[...])


def _dense_self1(x, W_r1, b_l1):
    return pl.pallas_call(
        _self1_body,
        grid=(GRID,),
        in_specs=[
            pl.BlockSpec((ROW_BLK, D_IN), lambda i: (i, 0)),
            pl.BlockSpec((D_IN, D_H), lambda i: (0, 0)),
            pl.BlockSpec((1, D_H), lambda i: (0, 0)),
        ],
        out_specs=pl.BlockSpec((ROW_BLK, D_H), lambda i: (i, 0)),
        out_shape=jax.ShapeDtypeStruct((N, D_H), jnp.float32),
    )(x, W_r1.T, b_l1[None, :])


def _sage1_body(sa_ref, sb_ref, dga_ref, dgb_ref, za_ref, wl_ref, h_ref):
    inv = 1.0 / jnp.maximum(dga_ref[...] + dgb_ref[...], 1.0)
    mean = (sa_ref[...] + sb_ref[...]) * inv
    z = jnp.dot(mean, wl_ref[...],
                preferred_element_type=jnp.float32) + za_ref[...]
    h = jnp.maximum(z, 0.0)
    h_ref[:, 0, :] = h[:, :D_IN]
    h_ref[:, 1, :] = h[:, D_IN:]


def _dense_layer1(parts, dega2d, degb2d, za, W_l1):
    return pl.pallas_call(
        _sage1_body,
        grid=(GRID,),
        in_specs=[
            pl.BlockSpec((ROW_BLK, D_IN), lambda i: (i, 0)),
            pl.BlockSpec((ROW_BLK, D_IN), lambda i: (i + GRID, 0)),
            pl.BlockSpec((ROW_BLK, 1), lambda i: (i, 0)),
            pl.BlockSpec((ROW_BLK, 1), lambda i: (i, 0)),
            pl.BlockSpec((ROW_BLK, D_H), lambda i: (i, 0)),
            pl.BlockSpec((D_IN, D_H), lambda i: (0, 0)),
        ],
        out_specs=pl.BlockSpec((ROW_BLK, 2, D_IN), lambda i: (i, 0, 0)),
        out_shape=jax.ShapeDtypeStruct((N, 2, D_IN), jnp.float32),
    )(parts, parts, dega2d, degb2d, za, W_l1.T)


def _self2_body(h_ref, wra_ref, wrb_ref, b_ref, z_ref):
    z_ref[...] = (
        jnp.dot(h_ref[:, 0, :], wra_ref[...],
                preferred_element_type=jnp.float32)
        + jnp.dot(h_ref[:, 1, :], wrb_ref[...],
                  preferred_element_type=jnp.float32)
        + b_ref[...])


def _dense_self2(h1, W_r2, b_l2):
    wr2 = W_r2.T
    return pl.pallas_call(
        _self2_body,
        grid=(GRID,),
        in_specs=[
            pl.BlockSpec((ROW_BLK, 2, D_IN), lambda i: (i, 0, 0)),
            pl.BlockSpec((D_IN, D_H), lambda i: (0, 0)),
            pl.BlockSpec((D_IN, D_H), lambda i: (0, 0)),
            pl.BlockSpec((1, D_H), lambda i: (0, 0)),
        ],
        out_specs=pl.BlockSpec((ROW_BLK, D_H), lambda i: (i, 0)),
        out_shape=jax.ShapeDtypeStruct((N, D_H), jnp.float32),
    )(h1, wr2[:D_IN], wr2[D_IN:], b_l2[None, :])


def _sage2_pool_body(sa_ref, sb_ref, dga_ref, dgb_ref, zb_ref, wla_ref,
                     wlb_ref, out_ref):
    inv = 1.0 / jnp.maximum(dga_ref[...] + dgb_ref[...], 1.0)
    mean_a = sa_ref[...] * inv
    mean_b = sb_ref[...] * inv
    z = (jnp.dot(mean_a, wla_ref[...], preferred_element_type=jnp.float32)
         + jnp.dot(mean_b, wlb_ref[...], preferred_element_type=jnp.float32)
         + zb_ref[...])
    h2 = jnp.maximum(z, 0.0)
    blk_sum = jnp.sum(h2, axis=0, keepdims=True)

    @pl.when(pl.program_id(0) == 0)
    def _init():
        out_ref[...] = jnp.zeros_like(out_ref)

    out_ref[...] += blk_sum * (1.0 / N)


def _dense_layer2_pool(summed2, dega2d, degb2d, zb, W_l2):
    wl2 = W_l2.T
    return pl.pallas_call(
        _sage2_pool_body,
        grid=(GRID,),
        in_specs=[
            pl.BlockSpec((ROW_BLK, D_IN), lambda i: (i, 0)),
            pl.BlockSpec((ROW_BLK, D_IN), lambda i: (i + GRID, 0)),
            pl.BlockSpec((ROW_BLK, 1), lambda i: (i, 0)),
            pl.BlockSpec((ROW_BLK, 1), lambda i: (i, 0)),
            pl.BlockSpec((ROW_BLK, D_H), lambda i: (i, 0)),
            pl.BlockSpec((D_IN, D_H), lambda i: (0, 0)),
            pl.BlockSpec((D_IN, D_H), lambda i: (0, 0)),
        ],
        out_specs=pl.BlockSpec((1, D_H), lambda i: (0, 0)),
        out_shape=jax.ShapeDtypeStruct((1, D_H), jnp.float32),
    )(summed2, summed2, dega2d, degb2d, zb, wl2[:D_IN], wl2[D_IN:])


def kernel(x, edge_index, batch, W_l1, b_l1, W_r1, W_l2, b_l2, W_r2):
    src = edge_index[0]
    dst = edge_index[1]
    srcs1 = src.reshape(EROWS, C)
    dsts1 = dst.reshape(EROWS, C)
    srcs2 = jnp.stack([2 * src, 2 * src + 1]).reshape(2, EROWS, C)

    zeros = jnp.zeros((N, D_IN), jnp.float32)
    zerosd = jnp.zeros((NPAD,), jnp.float32)

    (parts1,) = _sc_agg1(x, srcs1, dsts1, zeros)
    (degp,) = _sc_deg(dst, zerosd)
    za = _dense_self1(x, W_r1, b_l1)
    dega2d = degp[0, :N, None]
    degb2d = degp[1, :N, None]
    h1 = _dense_layer1(parts1, dega2d, degb2d, za, W_l1)

    ht = h1.reshape(2 * N, D_IN)
    (summed2,) = _sc_agg2(ht, srcs2, dsts1, zeros)
    zb = _dense_self2(h1, W_r2, b_l2)
    pooled = _dense_layer2_pool(summed2, dega2d, degb2d, zb, W_l2)
    return pooled[0]


# speedup vs baseline: 1.0090x; 1.0090x over previous
import functools

import jax
import jax.numpy as jnp
from jax import lax
from jax.experimental import pallas as pl
from jax.experimental.pallas import tpu as pltpu
from jax.experimental.pallas import tpu_sc as plsc

N = 10000
E = 320000
D_IN = 128
D_H = 256
NPAD = 10240

NC = 2
NS = 16
C = 125
EROWS = E // C
ROWS1 = EROWS // (NC * NS)
ROWS2 = EROWS // NS

GLEN = 16

ROW_BLK = 1000
GRID = N // ROW_BLK

_sc_mesh = plsc.VectorSubcoreMesh(
    core_axis_name="c", subcore_axis_name="s", num_cores=NC, num_subcores=NS)


@functools.partial(
    pl.kernel,
    out_type=[
        jax.ShapeDtypeStruct((2 * N, D_IN), jnp.float32),
    ],
    mesh=_sc_mesh,
    scratch_types=[
        pltpu.VMEM((2 * GLEN, C), jnp.int32),
        pltpu.VMEM((2 * GLEN, C), jnp.int32),
        pltpu.VMEM((C, D_IN), jnp.float32),
        pltpu.VMEM((C, D_IN), jnp.float32),
        pltpu.VMEM_SHARED((N, D_IN), jnp.float32),
        pltpu.SemaphoreType.DMA,
        pltpu.SemaphoreType.DMA,
        pltpu.SemaphoreType.DMA,
        pltpu.SemaphoreType.DMA,
        pltpu.SemaphoreType.DMA,
    ],
)
def _sc_agg1(x_hbm, srcs_hbm, dsts_hbm, zeros_hbm, out_hbm,
             src_v, dst_v, rows0, rows1, acc_sh,
             gsem0, gsem1, ssem0, ssem1, isem):
    c = lax.axis_index("c")
    s = lax.axis_index("s")

    @pl.when(s == 0)
    def _init():
        pltpu.sync_copy(zeros_hbm, acc_sh)

    base = c * (NS * ROWS1) + s * ROWS1
    pltpu.sync_copy(srcs_hbm.at[pl.ds(base, GLEN)], src_v.at[pl.ds(0, GLEN)])
    pltpu.sync_copy(dsts_hbm.at[pl.ds(base, GLEN)], dst_v.at[pl.ds(0, GLEN)])
    plsc.subcore_barrier()
    _agg_pipeline(x_hbm, srcs_hbm, dsts_hbm, base,
                  ROWS1 // GLEN, src_v, dst_v, (rows0, rows1),
                  (gsem0, gsem1), (ssem0, ssem1), isem, acc_sh)
    plsc.subcore_barrier()

    @pl.when(s == 0)
    def _writeout():
        pltpu.sync_copy(acc_sh, out_hbm.at[pl.ds(c * N, N)])


def _agg_pipeline(tab_hbm, srcs_ref, dsts_ref, base, ng, src_v, dst_v,
                  rows, gsem, ssem, isem, acc_sh):
    def swait(p):
        pltpu.make_async_copy(rows[p], acc_sh.at[dst_v.at[0]], ssem[p]).wait()

    def group(g, carry):
        gp = lax.rem(g, 2)
        row0 = gp * GLEN

        @pl.when(g + 1 < ng)
        def _prefetch_idx():
            pltpu.async_copy(srcs_ref.at[pl.ds(base + (g + 1) * GLEN, GLEN)],
                             src_v.at[pl.ds((1 - gp) * GLEN, GLEN)], isem)
            pltpu.async_copy(dsts_ref.at[pl.ds(base + (g + 1) * GLEN, GLEN)],
                             dst_v.at[pl.ds((1 - gp) * GLEN, GLEN)], isem)

        @pl.when(g > 0)
        def _drain0():
            swait(0)

        gd = [pltpu.async_copy(tab_hbm.at[src_v.at[row0]], rows[0], gsem[0]),
              None]
        for jj in range(GLEN):
            p = jj % 2
            if jj < GLEN - 1:
                if jj >= 1:
                    swait(1 - p)
                else:
                    @pl.when(g > 0)
                    def _drain1():
                        swait(1)
                gd[1 - p] = pltpu.async_copy(
                    tab_hbm.at[src_v.at[row0 + jj + 1]], rows[1 - p],
                    gsem[1 - p])
            gd[p].wait()
            pltpu.async_copy(
                rows[p], acc_sh.at[dst_v.at[row0 + jj]], ssem[p], add=True)

        @pl.when(g + 1 < ng)
        def _wait_idx():
            pltpu.make_async_copy(srcs_ref.at[pl.ds(base, GLEN)],
                                  src_v.at[pl.ds((1 - gp) * GLEN, GLEN)],
                                  isem).wait()
            pltpu.make_async_copy(dsts_ref.at[pl.ds(base, GLEN)],
                                  dst_v.at[pl.ds((1 - gp) * GLEN, GLEN)],
                                  isem).wait()
        return carry

    lax.fori_loop(0, ng, group, 0)
    swait(0)
    swait(1)


@functools.partial(
    pl.kernel,
    out_type=[jax.ShapeDtypeStruct((2, NPAD), jnp.float32)],
    mesh=_sc_mesh,
    compiler_params=pltpu.CompilerParams(needs_layout_passes=False),
    scratch_types=[
        pltpu.VMEM((4000,), jnp.int32),
        pltpu.VMEM((NPAD,), jnp.float32),
        pltpu.VMEM((NS * 320,), jnp.float32),
        pltpu.VMEM((640,), jnp.float32),
        pltpu.VMEM_SHARED((NS * NPAD,), jnp.float32),
    ],
)
def _sc_deg(dstf_hbm, zerosd_hbm, deg_hbm, dfl_v, deg_t, red_v, dout_v,
            degs_sh):
    c = lax.axis_index("c")
    s = lax.axis_index("s")
    pltpu.sync_copy(zerosd_hbm, deg_t)

    base = c * (E // 2) + s * 10000
    for part, size in ((0, 4000), (4000, 4000), (8000, 2000)):
        pltpu.sync_copy(dstf_hbm.at[pl.ds(base + part, size)],
                        dfl_v.at[pl.ds(0, size)])

        def hist(i, carry):
            idx16 = dfl_v[pl.ds(i * 16, 16)]
            cnt, last = plsc.scan_count(idx16)
            plsc.addupdate_scatter(
                deg_t, [idx16], cnt.astype(jnp.float32), mask=last)
            return carry

        lax.fori_loop(0, size // 16, hist, 0)
    pltpu.sync_copy(deg_t, degs_sh.at[pl.ds(s * NPAD, NPAD)])
    plsc.subcore_barrier()

    for half in range(2):
        col0 = s * 640 + half * 320
        for r in range(NS):
            pltpu.sync_copy(degs_sh.at[pl.ds(r * NPAD + col0, 320)],
                            red_v.at[pl.ds(r * 320, 320)])

        def red(k, carry):
            acc16 = red_v[pl.ds(k * 16, 16)]
            for r in range(1, NS):
                acc16 = acc16 + red_v[pl.ds(r * 320 + k * 16, 16)]
            dout_v[pl.ds(half * 320 + k * 16, 16)] = acc16
            return carry

        lax.fori_loop(0, 320 // 16, red, 0)
    pltpu.sync_copy(dout_v, deg_hbm.at[c, pl.ds(s * 640, 640)])


@functools.partial(
    pl.kernel,
    out_type=[
        jax.ShapeDtypeStruct((2 * N, D_IN), jnp.float32),
    ],
    mesh=_sc_mesh,
    scratch_types=[
        pltpu.VMEM((2 * GLEN, C), jnp.int32),
        pltpu.VMEM((2 * GLEN, C), jnp.int32),
        pltpu.VMEM((C, D_IN), jnp.float32),
        pltpu.VMEM((C, D_IN), jnp.float32),
        pltpu.VMEM_SHARED((N, D_IN), jnp.float32),
        pltpu.SemaphoreType.DMA,
        pltpu.SemaphoreType.DMA,
        pltpu.SemaphoreType.DMA,
        pltpu.SemaphoreType.DMA,
        pltpu.SemaphoreType.DMA,
    ],
)
def _sc_agg2(ht_hbm, srcs2_hbm, dsts_hbm, zeros_hbm, out_hbm,
             src_v, dst_v, rows0, rows1, acc_sh,
             gsem0, gsem1, ssem0, ssem1, isem):
    c = lax.axis_index("c")
    s = lax.axis_index("s")

    @pl.when(s == 0)
    def _init():
        pltpu.sync_copy(zeros_hbm, acc_sh)

    base = s * ROWS2
    srcs_ref = srcs2_hbm.at[c]
    pltpu.sync_copy(srcs_ref.at[pl.ds(base, GLEN)], src_v.at[pl.ds(0, GLEN)])
    pltpu.sync_copy(dsts_hbm.at[pl.ds(base, GLEN)], dst_v.at[pl.ds(0, GLEN)])
    plsc.subcore_barrier()
    _agg_pipeline(ht_hbm, srcs_ref, dsts_hbm, base,
                  ROWS2 // GLEN, src_v, dst_v, (rows0, rows1),
                  (gsem0, gsem1), (ssem0, ssem1), isem, acc_sh)
    plsc.subcore_barrier()

    @pl.when(s == 0)
    def _writeout():
        pltpu.sync_copy(acc_sh, out_hbm.at[pl.ds(c * N, N)])


def _sage1_body(sa_ref, sb_ref, dga_ref, dgb_ref, x_ref, wl_ref, b_ref,
                wr_ref, h_ref):
    inv = 1.0 / jnp.maximum(dga_ref[...] + dgb_ref[...], 1.0)
    mean = (sa_ref[...] + sb_ref[...]) * inv
    z = (jnp.dot(mean, wl_ref[...], preferred_element_type=jnp.float32)
         + jnp.dot(x_ref[...], wr_ref[...], preferred_element_type=jnp.float32)
         + b_ref[...])
    h = jnp.maximum(z, 0.0)
    h_ref[:, 0, :] = h[:, :D_IN]
    h_ref[:, 1, :] = h[:, D_IN:]


def _dense_layer1(parts, dega2d, degb2d, x, W_l1, b_l1, W_r1):
    return pl.pallas_call(
        _sage1_body,
        grid=(GRID,),
        in_specs=[
            pl.BlockSpec((ROW_BLK, D_IN), lambda i: (i, 0)),
            pl.BlockSpec((ROW_BLK, D_IN), lambda i: (i + GRID, 0)),
            pl.BlockSpec((ROW_BLK, 1), lambda i: (i, 0)),
            pl.BlockSpec((ROW_BLK, 1), lambda i: (i, 0)),
            pl.BlockSpec((ROW_BLK, D_IN), lambda i: (i, 0)),
            pl.BlockSpec((D_IN, D_H), lambda i: (0, 0)),
            pl.BlockSpec((1, D_H), lambda i: (0, 0)),
            pl.BlockSpec((D_IN, D_H), lambda i: (0, 0)),
        ],
        out_specs=pl.BlockSpec((ROW_BLK, 2, D_IN), lambda i: (i, 0, 0)),
        out_shape=jax.ShapeDtypeStruct((N, 2, D_IN), jnp.float32),
    )(parts, parts, dega2d, degb2d, x, W_l1.T, b_l1[None, :], W_r1.T)


def _sage2_pool_body(sa_ref, sb_ref, dga_ref, dgb_ref, h_ref, wla_ref,
                     wlb_ref, b_ref, wra_ref, wrb_ref, out_ref):
    inv = 1.0 / jnp.maximum(dga_ref[...] + dgb_ref[...], 1.0)
    mean_a = sa_ref[...] * inv
    mean_b = sb_ref[...] * inv
    h1a = h_ref[:, 0, :]
    h1b = h_ref[:, 1, :]
    z = (jnp.dot(mean_a, wla_ref[...], preferred_element_type=jnp.float32)
         + jnp.dot(mean_b, wlb_ref[...], preferred_element_type=jnp.float32)
         + jnp.dot(h1a, wra_ref[...], preferred_element_type=jnp.float32)
         + jnp.dot(h1b, wrb_ref[...], preferred_element_type=jnp.float32)
         + b_ref[...])
    h2 = jnp.maximum(z, 0.0)
    blk_sum = jnp.sum(h2, axis=0, keepdims=True)

    @pl.when(pl.program_id(0) == 0)
    def _init():
        out_ref[...] = jnp.zeros_like(out_ref)

    out_ref[...] += blk_sum * (1.0 / N)


def _dense_layer2_pool(summed2, dega2d, degb2d, h1, W_l2, b_l2, W_r2):
    wl2 = W_l2.T
    wr2 = W_r2.T
    return pl.pallas_call(
        _sage2_pool_body,
        grid=(GRID,),
        in_specs=[
            pl.BlockSpec((ROW_BLK, D_IN), lambda i: (i, 0)),
            pl.BlockSpec((ROW_BLK, D_IN), lambda i: (i + GRID, 0)),
            pl.BlockSpec((ROW_BLK, 1), lambda i: (i, 0)),
            pl.BlockSpec((ROW_BLK, 1), lambda i: (i, 0)),
            pl.BlockSpec((ROW_BLK, 2, D_IN), lambda i: (i, 0, 0)),
            pl.BlockSpec((D_IN, D_H), lambda i: (0, 0)),
            pl.BlockSpec((D_IN, D_H), lambda i: (0, 0)),
            pl.BlockSpec((1, D_H), lambda i: (0, 0)),
            pl.BlockSpec((D_IN, D_H), lambda i: (0, 0)),
            pl.BlockSpec((D_IN, D_H), lambda i: (0, 0)),
        ],
        out_specs=pl.BlockSpec((1, D_H), lambda i: (0, 0)),
        out_shape=jax.ShapeDtypeStruct((1, D_H), jnp.float32),
    )(summed2, summed2, dega2d, degb2d, h1,
      wl2[:D_IN], wl2[D_IN:], b_l2[None, :], wr2[:D_IN], wr2[D_IN:])


def kernel(x, edge_index, batch, W_l1, b_l1, W_r1, W_l2, b_l2, W_r2):
    src = edge_index[0]
    dst = edge_index[1]
    srcs1 = src.reshape(EROWS, C)
    dsts1 = dst.reshape(EROWS, C)
    srcs2 = jnp.stack([2 * src, 2 * src + 1]).reshape(2, EROWS, C)

    zeros = jnp.zeros((N, D_IN), jnp.float32)
    zerosd = jnp.zeros((NPAD,), jnp.float32)

    (parts1,) = _sc_agg1(x, srcs1, dsts1, zeros)
    (degp,) = _sc_deg(dst, zerosd)
    dega2d = degp[0, :N, None]
    degb2d = degp[1, :N, None]
    h1 = _dense_layer1(parts1, dega2d, degb2d, x, W_l1, b_l1, W_r1)

    ht = h1.reshape(2 * N, D_IN)
    (summed2,) = _sc_agg2(ht, srcs2, dsts1, zeros)
    pooled = _dense_layer2_pool(summed2, dega2d, degb2d, h1, W_l2, b_l2, W_r2)
    return pooled[0]
